# baseline (speedup 1.0000x reference)
import jax
import jax.numpy as jnp
from jax.experimental import pallas as pl
from jax.experimental.pallas import tpu as pltpu

TOKEN_BLOCK = 512


def _router_block(xa_ref, xb_ref, w_ref, oa_ref, ob_ref):
    w = w_ref[...]
    oa_ref[...] = jax.nn.sigmoid(
        jnp.dot(xa_ref[...], w, preferred_element_type=jnp.float32))
    ob_ref[...] = jax.nn.sigmoid(
        jnp.dot(xb_ref[...], w, preferred_element_type=jnp.float32))


@jax.jit
def kernel(x, router_weight):
    tokens, dim = x.shape
    num_experts = router_weight.shape[0]
    wt = router_weight.T

    half = tokens // 2
    nblk = half // TOKEN_BLOCK
    oa, ob = pl.pallas_call(
        _router_block,
        grid=(nblk,),
        in_specs=[
            pl.BlockSpec((TOKEN_BLOCK, dim), lambda i: (i, 0)),
            pl.BlockSpec((TOKEN_BLOCK, dim), lambda i: (i + nblk, 0)),
            pl.BlockSpec((dim, num_experts), lambda i: (0, 0)),
        ],
        out_specs=[
            pl.BlockSpec((TOKEN_BLOCK, num_experts), lambda i: (i, 0)),
            pl.BlockSpec((TOKEN_BLOCK, num_experts), lambda i: (i, 0)),
        ],
        out_shape=[
            jax.ShapeDtypeStruct((half, num_experts), jnp.float32),
            jax.ShapeDtypeStruct((half, num_experts), jnp.float32),
        ],
        compiler_params=pltpu.CompilerParams(
            dimension_semantics=("parallel",),
        ),
    )(x, x, wt)
    return jnp.concatenate([oa, ob], axis=0)


# speedup vs baseline: 1.1460x; 1.1460x over previous
import jax
import jax.numpy as jnp
from jax.experimental import pallas as pl
from jax.experimental.pallas import tpu as pltpu

TOKEN_BLOCK = 512


def _router_block(x_ref, w_ref, out_ref):
    logits = jnp.dot(x_ref[...], w_ref[...], preferred_element_type=jnp.float32)
    out_ref[...] = jax.nn.sigmoid(logits)


@jax.jit
def kernel(x, router_weight):
    tokens, dim = x.shape
    num_experts = router_weight.shape[0]
    wt = router_weight.T

    grid = (tokens // TOKEN_BLOCK,)
    return pl.pallas_call(
        _router_block,
        grid=grid,
        in_specs=[
            pl.BlockSpec((TOKEN_BLOCK, dim), lambda i: (i, 0)),
            pl.BlockSpec((dim, num_experts), lambda i: (0, 0)),
        ],
        out_specs=pl.BlockSpec((TOKEN_BLOCK, num_experts), lambda i: (i, 0)),
        out_shape=jax.ShapeDtypeStruct((tokens, num_experts), jnp.float32),
        compiler_params=pltpu.CompilerParams(
            dimension_semantics=("parallel",),
        ),
    )(x, wt)
